# baseline (speedup 1.0000x reference)
import functools

import jax
import jax.numpy as jnp
from jax import lax
from jax.experimental import pallas as pl
from jax.experimental.pallas import tpu as pltpu

T = 2048
D = 2048
FF = 2048
E = 8
TOPK = 2
M = T * TOPK

BM = 256
M_BLOCKS = M // BM
NUM_ITEMS = M_BLOCKS + E - 1
BF = 1024
N_FF = FF // BF


def _routing_metadata(flat_experts, sorted_idx):
    counts = jnp.bincount(flat_experts, length=E)
    offsets = jnp.concatenate([jnp.zeros((1,), jnp.int32),
                               jnp.cumsum(counts).astype(jnp.int32)])
    b_grid = jnp.arange(M_BLOCKS, dtype=jnp.int32)[:, None]
    e_grid = jnp.arange(E, dtype=jnp.int32)[None, :]
    lo_g = jnp.maximum(offsets[:-1][None, :], b_grid * BM)
    hi_g = jnp.minimum(offsets[1:][None, :], (b_grid + 1) * BM)
    valid = lo_g < hi_g
    key = jnp.where(valid, b_grid * E + e_grid, 1 << 30).reshape(-1)
    order = jnp.argsort(key)[:NUM_ITEMS]
    kf = key[order]
    pad = kf >= (1 << 30)
    b_arr = jnp.where(pad, M_BLOCKS - 1, kf // E).astype(jnp.int32)
    e_arr = jnp.where(pad, E - 1, kf % E).astype(jnp.int32)
    lo_arr = jnp.where(pad, 0, lo_g.reshape(-1)[order] - b_arr * BM).astype(jnp.int32)
    hi_arr = jnp.where(pad, 0, hi_g.reshape(-1)[order] - b_arr * BM).astype(jnp.int32)
    first = jnp.concatenate([jnp.ones((1,), jnp.int32),
                             (b_arr[1:] != b_arr[:-1]).astype(jnp.int32)])
    return b_arr, e_arr, lo_arr, hi_arr, first


def _mlp_body(b_ref, e_ref, lo_ref, hi_ref, first_ref, x_ref, w1a_ref, w1b_ref,
              w2_ref, s_ref, y_ref):
    i = pl.program_id(0)
    lo = lo_ref[i]
    hi = hi_ref[i]
    first = first_ref[i]

    @pl.when(hi > lo)
    def _():
        x = x_ref[...]
        p = jnp.dot(x, w1a_ref[0], preferred_element_type=jnp.float32)
        g = jnp.dot(x, w1b_ref[0], preferred_element_type=jnp.float32)
        h = jax.nn.silu(p) * g
        rows = lax.broadcasted_iota(jnp.int32, (BM, FF), 0)
        h = jnp.where((rows >= lo) & (rows < hi), h, 0.0).astype(jnp.bfloat16)
        y = jnp.dot(h, w2_ref[0], preferred_element_type=jnp.float32)
        val = y * s_ref[...]

        @pl.when(first == 1)
        def _():
            y_ref[...] = val

        @pl.when(first == 0)
        def _():
            y_ref[...] += val


def _grouped_mlp(meta, xs, W1, W2, s_sorted, interpret=False):
    b_arr, e_arr, lo_arr, hi_arr, first = meta
    mlp = pl.pallas_call(
        _mlp_body,
        grid_spec=pltpu.PrefetchScalarGridSpec(
            num_scalar_prefetch=5,
            grid=(NUM_ITEMS,),
            in_specs=[
                pl.BlockSpec((BM, D), lambda i, b, e, lo, hi, fs: (b[i], 0)),
                pl.BlockSpec((1, D, FF),
                             lambda i, b, e, lo, hi, fs: (e[i], 0, 0)),
                pl.BlockSpec((1, D, FF),
                             lambda i, b, e, lo, hi, fs: (e[i], 0, 1)),
                pl.BlockSpec((1, FF, D),
                             lambda i, b, e, lo, hi, fs: (e[i], 0, 0)),
                pl.BlockSpec((BM, 1), lambda i, b, e, lo, hi, fs: (b[i], 0)),
            ],
            out_specs=pl.BlockSpec((BM, D),
                                   lambda i, b, e, lo, hi, fs: (b[i], 0)),
        ),
        out_shape=jax.ShapeDtypeStruct((M, D), jnp.float32),
        interpret=interpret,
    )
    return mlp(b_arr, e_arr, lo_arr, hi_arr, first, xs, W1, W1, W2, s_sorted)


def kernel(hidden_states, router_logits, W1, W2):
    top_logits, top_indices = lax.top_k(router_logits, TOPK)
    scores = jax.nn.softmax(top_logits, axis=-1)
    flat = top_indices.reshape(-1).astype(jnp.int32)
    sorted_idx = jnp.argsort(flat, stable=True).astype(jnp.int32)
    meta = _routing_metadata(flat, sorted_idx)

    xs = hidden_states.astype(jnp.bfloat16)[sorted_idx // TOPK]
    s_sorted = scores.reshape(-1)[sorted_idx][:, None]

    ys = _grouped_mlp(meta, xs, W1.astype(jnp.bfloat16),
                      W2.astype(jnp.bfloat16), s_sorted)

    unperm = jnp.zeros((M, D), jnp.float32).at[sorted_idx].set(ys)
    return unperm.reshape(T, TOPK, D).sum(axis=1)


# speedup vs baseline: 1.1880x; 1.1880x over previous
import functools

import jax
import jax.numpy as jnp
from jax import lax
from jax.experimental import pallas as pl
from jax.experimental.pallas import tpu as pltpu

T = 2048
D = 2048
FF = 2048
E = 8
TOPK = 2
M = T * TOPK

BM = 256
M_BLOCKS = M // BM
NUM_ITEMS = M_BLOCKS + E - 1
BF = 1024
N_FF = FF // BF


def _routing_metadata(flat_experts, sorted_idx):
    counts = jnp.bincount(flat_experts, length=E)
    offsets = jnp.concatenate([jnp.zeros((1,), jnp.int32),
                               jnp.cumsum(counts).astype(jnp.int32)])
    b_grid = jnp.arange(M_BLOCKS, dtype=jnp.int32)[:, None]
    e_grid = jnp.arange(E, dtype=jnp.int32)[None, :]
    lo_g = jnp.maximum(offsets[:-1][None, :], b_grid * BM)
    hi_g = jnp.minimum(offsets[1:][None, :], (b_grid + 1) * BM)
    valid = lo_g < hi_g
    key = jnp.where(valid, b_grid * E + e_grid, 1 << 30).reshape(-1)
    order = jnp.argsort(key)[:NUM_ITEMS]
    kf = key[order]
    pad = kf >= (1 << 30)
    b_arr = jnp.where(pad, M_BLOCKS - 1, kf // E).astype(jnp.int32)
    e_arr = jnp.where(pad, E - 1, kf % E).astype(jnp.int32)
    lo_arr = jnp.where(pad, 0, lo_g.reshape(-1)[order] - b_arr * BM).astype(jnp.int32)
    hi_arr = jnp.where(pad, 0, hi_g.reshape(-1)[order] - b_arr * BM).astype(jnp.int32)
    first = jnp.concatenate([jnp.ones((1,), jnp.int32),
                             (b_arr[1:] != b_arr[:-1]).astype(jnp.int32)])
    return b_arr, e_arr, lo_arr, hi_arr, first


def _fc1_body(b_ref, e_ref, lo_ref, hi_ref, first_ref, x_ref, w1a_ref, w1b_ref,
              h_ref):
    i = pl.program_id(1)
    lo = lo_ref[i]
    hi = hi_ref[i]
    first = first_ref[i]

    @pl.when(hi > lo)
    def _():
        x = x_ref[...]
        w1a = w1a_ref[0].astype(jnp.bfloat16)
        w1b = w1b_ref[0].astype(jnp.bfloat16)
        p = jnp.dot(x, w1a, preferred_element_type=jnp.float32)
        g = jnp.dot(x, w1b, preferred_element_type=jnp.float32)
        val = jax.nn.silu(p) * g
        rows = lax.broadcasted_iota(jnp.int32, (BM, BF), 0)
        val = jnp.where((rows >= lo) & (rows < hi), val, 0.0).astype(jnp.bfloat16)

        @pl.when(first == 1)
        def _():
            h_ref[...] = val

        @pl.when(first == 0)
        def _():
            h_ref[...] += val


def _fc2_body(b_ref, e_ref, lo_ref, hi_ref, first_ref, h_ref, w2_ref, s_ref,
              y_ref):
    i = pl.program_id(0)
    lo = lo_ref[i]
    hi = hi_ref[i]
    first = first_ref[i]

    @pl.when(hi > lo)
    def _():
        w2 = w2_ref[0].astype(jnp.bfloat16)
        y = jnp.dot(h_ref[...], w2, preferred_element_type=jnp.float32)
        y = y * s_ref[...]
        rows = lax.broadcasted_iota(jnp.int32, (BM, D), 0)
        val = jnp.where((rows >= lo) & (rows < hi), y, 0.0)

        @pl.when(first == 1)
        def _():
            y_ref[...] = val

        @pl.when(first == 0)
        def _():
            y_ref[...] += val


def _grouped_mlp(meta, xs, W1, W2, s_sorted, interpret=False):
    b_arr, e_arr, lo_arr, hi_arr, first = meta
    fc1 = pl.pallas_call(
        _fc1_body,
        grid_spec=pltpu.PrefetchScalarGridSpec(
            num_scalar_prefetch=5,
            grid=(N_FF, NUM_ITEMS),
            in_specs=[
                pl.BlockSpec((BM, D), lambda j, i, b, e, lo, hi, fs: (b[i], 0)),
                pl.BlockSpec((1, D, BF),
                             lambda j, i, b, e, lo, hi, fs: (e[i], 0, j)),
                pl.BlockSpec((1, D, BF),
                             lambda j, i, b, e, lo, hi, fs: (e[i], 0, N_FF + j)),
            ],
            out_specs=pl.BlockSpec((BM, BF),
                                   lambda j, i, b, e, lo, hi, fs: (b[i], j)),
        ),
        out_shape=jax.ShapeDtypeStruct((M, FF), jnp.bfloat16),
        interpret=interpret,
    )
    h = fc1(b_arr, e_arr, lo_arr, hi_arr, first, xs, W1, W1)
    fc2 = pl.pallas_call(
        _fc2_body,
        grid_spec=pltpu.PrefetchScalarGridSpec(
            num_scalar_prefetch=5,
            grid=(NUM_ITEMS,),
            in_specs=[
                pl.BlockSpec((BM, FF), lambda i, b, e, lo, hi, fs: (b[i], 0)),
                pl.BlockSpec((1, FF, D), lambda i, b, e, lo, hi, fs: (e[i], 0, 0)),
                pl.BlockSpec((BM, 1), lambda i, b, e, lo, hi, fs: (b[i], 0)),
            ],
            out_specs=pl.BlockSpec((BM, D),
                                   lambda i, b, e, lo, hi, fs: (b[i], 0)),
        ),
        out_shape=jax.ShapeDtypeStruct((M, D), jnp.float32),
        interpret=interpret,
    )
    return fc2(b_arr, e_arr, lo_arr, hi_arr, first, h, W2, s_sorted)


def kernel(hidden_states, router_logits, W1, W2):
    top_logits, top_indices = lax.top_k(router_logits, TOPK)
    scores = jax.nn.softmax(top_logits, axis=-1)
    flat = top_indices.reshape(-1).astype(jnp.int32)
    sorted_idx = jnp.argsort(flat, stable=True).astype(jnp.int32)
    meta = _routing_metadata(flat, sorted_idx)

    xs = hidden_states.astype(jnp.bfloat16)[sorted_idx // TOPK]
    s_sorted = scores.reshape(-1)[sorted_idx][:, None]

    ys = _grouped_mlp(meta, xs, W1, W2, s_sorted)

    unperm = jnp.zeros((M, D), jnp.float32).at[sorted_idx].set(ys)
    return unperm.reshape(T, TOPK, D).sum(axis=1)


# speedup vs baseline: 1.4665x; 1.2344x over previous
import functools

import jax
import jax.numpy as jnp
from jax import lax
from jax.experimental import pallas as pl
from jax.experimental.pallas import tpu as pltpu
from jax.experimental.pallas import tpu_sc as plsc

T = 2048
D = 2048
FF = 2048
E = 8
TOPK = 2
M = T * TOPK

BM = 256
M_BLOCKS = M // BM
NUM_ITEMS = M_BLOCKS + E - 1
BF = 1024
N_FF = FF // BF


def _routing_metadata(flat_experts, sorted_idx):
    counts = jnp.bincount(flat_experts, length=E)
    offsets = jnp.concatenate([jnp.zeros((1,), jnp.int32),
                               jnp.cumsum(counts).astype(jnp.int32)])
    b_grid = jnp.arange(M_BLOCKS, dtype=jnp.int32)[:, None]
    e_grid = jnp.arange(E, dtype=jnp.int32)[None, :]
    lo_g = jnp.maximum(offsets[:-1][None, :], b_grid * BM)
    hi_g = jnp.minimum(offsets[1:][None, :], (b_grid + 1) * BM)
    valid = lo_g < hi_g
    key = jnp.where(valid, b_grid * E + e_grid, 1 << 30).reshape(-1)
    order = jnp.argsort(key)[:NUM_ITEMS]
    kf = key[order]
    pad = kf >= (1 << 30)
    b_arr = jnp.where(pad, M_BLOCKS - 1, kf // E).astype(jnp.int32)
    e_arr = jnp.where(pad, E - 1, kf % E).astype(jnp.int32)
    lo_arr = jnp.where(pad, 0, lo_g.reshape(-1)[order] - b_arr * BM).astype(jnp.int32)
    hi_arr = jnp.where(pad, 0, hi_g.reshape(-1)[order] - b_arr * BM).astype(jnp.int32)
    first = jnp.concatenate([jnp.ones((1,), jnp.int32),
                             (b_arr[1:] != b_arr[:-1]).astype(jnp.int32)])
    return b_arr, e_arr, lo_arr, hi_arr, first


_SC_INFO = plsc.get_sparse_core_info()
_NC = _SC_INFO.num_cores
_NS = _SC_INFO.num_subcores
_NW = _NC * _NS
_LANES = _SC_INFO.num_lanes

_GPW = M // _NW
_GCH = 32
_TPW = T // _NW
_CCH = 16


def _sc_mesh():
    return plsc.VectorSubcoreMesh(core_axis_name="c", subcore_axis_name="s")


def _sc_wid():
    return lax.axis_index("s") * _NC + lax.axis_index("c")


def _sc_gather(hidden_states, gidx):
    @functools.partial(
        pl.kernel,
        mesh=_sc_mesh(),
        out_type=jax.ShapeDtypeStruct((M, D), jnp.float32),
        scratch_types=[
            pltpu.VMEM((_GPW,), jnp.int32),
            pltpu.VMEM((_GCH, D), jnp.float32),
            pltpu.SemaphoreType.DMA,
        ],
    )
    def k(hid_hbm, gidx_hbm, xs_hbm, idx_v, rows_v, sem):
        base = _sc_wid() * _GPW
        pltpu.sync_copy(gidx_hbm.at[pl.ds(base, _GPW)], idx_v)
        for c in range(_GPW // _GCH):
            pltpu.async_copy(hid_hbm.at[idx_v.at[pl.ds(c * _GCH, _GCH)]],
                             rows_v, sem).wait()
            pltpu.sync_copy(rows_v, xs_hbm.at[pl.ds(base + c * _GCH, _GCH)])

    return k(hidden_states, gidx)


def _sc_combine(ys, i0, i1):
    @functools.partial(
        pl.kernel,
        mesh=_sc_mesh(),
        out_type=jax.ShapeDtypeStruct((T, D), jnp.float32),
        scratch_types=[
            pltpu.VMEM((_TPW,), jnp.int32),
            pltpu.VMEM((_TPW,), jnp.int32),
            pltpu.VMEM((_CCH, D), jnp.float32),
            pltpu.VMEM((_CCH, D), jnp.float32),
            pltpu.SemaphoreType.DMA,
        ],
    )
    def k(ys_hbm, i0_hbm, i1_hbm, out_hbm, i0_v, i1_v, a_v, b_v, sem):
        base = _sc_wid() * _TPW
        pltpu.sync_copy(i0_hbm.at[pl.ds(base, _TPW)], i0_v)
        pltpu.sync_copy(i1_hbm.at[pl.ds(base, _TPW)], i1_v)
        for c in range(_TPW // _CCH):
            pltpu.async_copy(ys_hbm.at[i0_v.at[pl.ds(c * _CCH, _CCH)]],
                             a_v, sem).wait()
            pltpu.async_copy(ys_hbm.at[i1_v.at[pl.ds(c * _CCH, _CCH)]],
                             b_v, sem).wait()
            for r in range(_CCH):
                def body(q, _, r=r):
                    sl = pl.ds(q * _LANES, _LANES)
                    a_v[r, sl] = a_v[r, sl] + b_v[r, sl]
                    return 0
                lax.fori_loop(0, D // _LANES, body, 0, unroll=8)
            pltpu.sync_copy(a_v, out_hbm.at[pl.ds(base + c * _CCH, _CCH)])

    return k(ys, i0, i1)


def _fc1_body(b_ref, e_ref, lo_ref, hi_ref, first_ref, x_ref, w1a_ref, w1b_ref,
              h_ref):
    i = pl.program_id(1)
    lo = lo_ref[i]
    hi = hi_ref[i]
    first = first_ref[i]

    @pl.when(hi > lo)
    def _():
        x = x_ref[...].astype(jnp.bfloat16)
        w1a = w1a_ref[0].astype(jnp.bfloat16)
        w1b = w1b_ref[0].astype(jnp.bfloat16)
        p = jnp.dot(x, w1a, preferred_element_type=jnp.float32)
        g = jnp.dot(x, w1b, preferred_element_type=jnp.float32)
        val = jax.nn.silu(p) * g
        rows = lax.broadcasted_iota(jnp.int32, (BM, BF), 0)
        val = jnp.where((rows >= lo) & (rows < hi), val, 0.0).astype(jnp.bfloat16)

        @pl.when(first == 1)
        def _():
            h_ref[...] = val

        @pl.when(first == 0)
        def _():
            h_ref[...] += val


def _fc2_body(b_ref, e_ref, lo_ref, hi_ref, first_ref, h_ref, w2_ref, s_ref,
              y_ref):
    i = pl.program_id(0)
    lo = lo_ref[i]
    hi = hi_ref[i]
    first = first_ref[i]

    @pl.when(hi > lo)
    def _():
        w2 = w2_ref[0].astype(jnp.bfloat16)
        y = jnp.dot(h_ref[...], w2, preferred_element_type=jnp.float32)
        y = y * s_ref[...]
        rows = lax.broadcasted_iota(jnp.int32, (BM, D), 0)
        val = jnp.where((rows >= lo) & (rows < hi), y, 0.0)

        @pl.when(first == 1)
        def _():
            y_ref[...] = val

        @pl.when(first == 0)
        def _():
            y_ref[...] += val


def _grouped_mlp(meta, xs, W1, W2, s_sorted, interpret=False):
    b_arr, e_arr, lo_arr, hi_arr, first = meta
    fc1 = pl.pallas_call(
        _fc1_body,
        grid_spec=pltpu.PrefetchScalarGridSpec(
            num_scalar_prefetch=5,
            grid=(N_FF, NUM_ITEMS),
            in_specs=[
                pl.BlockSpec((BM, D), lambda j, i, b, e, lo, hi, fs: (b[i], 0)),
                pl.BlockSpec((1, D, BF),
                             lambda j, i, b, e, lo, hi, fs: (e[i], 0, j)),
                pl.BlockSpec((1, D, BF),
                             lambda j, i, b, e, lo, hi, fs: (e[i], 0, N_FF + j)),
            ],
            out_specs=pl.BlockSpec((BM, BF),
                                   lambda j, i, b, e, lo, hi, fs: (b[i], j)),
        ),
        out_shape=jax.ShapeDtypeStruct((M, FF), jnp.bfloat16),
        interpret=interpret,
    )
    h = fc1(b_arr, e_arr, lo_arr, hi_arr, first, xs, W1, W1)
    fc2 = pl.pallas_call(
        _fc2_body,
        grid_spec=pltpu.PrefetchScalarGridSpec(
            num_scalar_prefetch=5,
            grid=(NUM_ITEMS,),
            in_specs=[
                pl.BlockSpec((BM, FF), lambda i, b, e, lo, hi, fs: (b[i], 0)),
                pl.BlockSpec((1, FF, D), lambda i, b, e, lo, hi, fs: (e[i], 0, 0)),
                pl.BlockSpec((BM, 1), lambda i, b, e, lo, hi, fs: (b[i], 0)),
            ],
            out_specs=pl.BlockSpec((BM, D),
                                   lambda i, b, e, lo, hi, fs: (b[i], 0)),
        ),
        out_shape=jax.ShapeDtypeStruct((M, D), jnp.float32),
        interpret=interpret,
    )
    return fc2(b_arr, e_arr, lo_arr, hi_arr, first, h, W2, s_sorted)


def kernel(hidden_states, router_logits, W1, W2):
    top_logits, top_indices = lax.top_k(router_logits, TOPK)
    scores = jax.nn.softmax(top_logits, axis=-1)
    flat = top_indices.reshape(-1).astype(jnp.int32)
    sorted_idx = jnp.argsort(flat, stable=True).astype(jnp.int32)
    meta = _routing_metadata(flat, sorted_idx)

    xs = _sc_gather(hidden_states, sorted_idx // TOPK)
    s_sorted = scores.reshape(-1)[sorted_idx]

    ys = _grouped_mlp(meta, xs, W1, W2, s_sorted[:, None])

    inv = jnp.zeros((M,), jnp.int32).at[sorted_idx].set(
        jnp.arange(M, dtype=jnp.int32))
    inv2 = inv.reshape(T, TOPK)
    return _sc_combine(ys, inv2[:, 0], inv2[:, 1])
